# baseline (speedup 1.0000x reference)
import functools

import jax
import jax.numpy as jnp
from jax import lax
from jax.experimental import pallas as pl
from jax.experimental.pallas import tpu as pltpu
from jax.experimental.pallas import tpu_sc as plsc

VOCAB_SIZE = 64
HIDDEN_DIM = 64

_LANE = 128
_ROWS = 16384 * 200 // _LANE
_SC_ROWS = 8704
_TC_ROWS = _ROWS - _SC_ROWS

_K = 4
_NW = 32
_ROWS_PER_W = _SC_ROWS // _NW
_OUTER = _ROWS_PER_W // _K


def _sc_body(x_hbm, table_hbm, out_hbm, idx_v, rows_v, table_s,
             sem_idx, sem_g, sem_out):
    wid = lax.axis_index("s") * 2 + lax.axis_index("c")
    w_base = wid * _ROWS_PER_W

    @pl.when(lax.axis_index("s") == 0)
    def _():
        pltpu.sync_copy(table_hbm, table_s)
    plsc.subcore_barrier()

    def idx_copy(slot, base):
        return pltpu.make_async_copy(
            x_hbm.at[pl.ds(base, _K)], idx_v.at[slot], sem_idx)

    def out_copy(slot, base):
        return pltpu.make_async_copy(
            rows_v.at[slot], out_hbm.at[pl.ds(base, _K)], sem_out)

    idx_copy(0, w_base).start()
    idx_copy(1, w_base + _K).start()

    def body(i, carry):
        for u in range(4):
            it = i * 4 + u
            rb = u % 2
            sb = u
            base = w_base + it * _K
            idx_copy(sb, base).wait()

            @pl.when(it >= 2)
            def _():
                out_copy(rb, base - 2 * _K).wait()

            gathers = [
                pltpu.async_copy(
                    table_s.at[idx_v.at[sb, j]], rows_v.at[rb, j], sem_g)
                for j in range(_K)
            ]

            @pl.when(it + 2 < _OUTER)
            def _():
                idx_copy((u + 2) % 4, base + 2 * _K).start()

            for g in gathers:
                g.wait()
            out_copy(rb, base).start()
        return carry

    lax.fori_loop(0, _OUTER // 4, body, 0)
    out_copy(0, w_base + (_OUTER - 2) * _K).wait()
    out_copy(1, w_base + (_OUTER - 1) * _K).wait()


def _run_sc(x2, table):
    mesh = plsc.VectorSubcoreMesh(core_axis_name="c", subcore_axis_name="s")
    return functools.partial(
        pl.kernel,
        mesh=mesh,
        out_type=jax.ShapeDtypeStruct((_SC_ROWS, _LANE, HIDDEN_DIM),
                                      jnp.float32),
        scratch_types=[
            pltpu.VMEM((4, _K, _LANE), jnp.int32),
            pltpu.VMEM((2, _K, _LANE, HIDDEN_DIM), jnp.float32),
            pltpu.VMEM_SHARED((VOCAB_SIZE, HIDDEN_DIM), jnp.float32),
            pltpu.SemaphoreType.DMA,
            pltpu.SemaphoreType.DMA,
            pltpu.SemaphoreType.DMA,
        ],
        compiler_params=pltpu.CompilerParams(use_tc_tiling_on_sc=False),
    )(_sc_body)(x2, table)


_TL = 1024
_SR = 32
_TC_N = _TC_ROWS * _LANE
_TC_XROWS = _TC_N // _TL


def _tc_body(x_ref, hi_ref, o_ref):
    dn = (((0,), (0,)), ((), ()))
    iota = jax.lax.broadcasted_iota(jnp.int32, (VOCAB_SIZE, _TL), 0)
    for j in range(_SR):
        oh = (x_ref[j, :][None, :] == iota).astype(jnp.bfloat16)
        acc = jax.lax.dot_general(oh, hi_ref[...], dn,
                                  preferred_element_type=jnp.float32)
        o_ref[pl.ds(j * _TL, _TL), :] = acc


def _run_tc(x2, table):
    xf = x2.reshape(_TC_XROWS, _TL)
    hi = table.astype(jnp.bfloat16)
    return pl.pallas_call(
        _tc_body,
        grid=(_TC_XROWS // _SR,),
        in_specs=[
            pl.BlockSpec((_SR, _TL), lambda i: (i, 0)),
            pl.BlockSpec((VOCAB_SIZE, HIDDEN_DIM), lambda i: (0, 0)),
        ],
        out_specs=pl.BlockSpec((_SR * _TL, HIDDEN_DIM), lambda i: (i, 0)),
        out_shape=jax.ShapeDtypeStruct((_TC_N, HIDDEN_DIM), jnp.float32),
    )(xf, hi)


def kernel(x, table):
    x2 = x.reshape(_ROWS, _LANE).astype(jnp.int32)
    sc_out = _run_sc(x2[:_SC_ROWS], table)
    tc_out = _run_tc(x2[_SC_ROWS:], table)
    out = jnp.concatenate(
        [sc_out.reshape(_SC_ROWS * _LANE, HIDDEN_DIM), tc_out], axis=0)
    return out.reshape(16384, 200, HIDDEN_DIM)


# speedup vs baseline: 1.0782x; 1.0782x over previous
import functools

import jax
import jax.numpy as jnp
from jax import lax
from jax.experimental import pallas as pl
from jax.experimental.pallas import tpu as pltpu
from jax.experimental.pallas import tpu_sc as plsc

VOCAB_SIZE = 64
HIDDEN_DIM = 64

_LANE = 128
_ROWS = 16384 * 200 // _LANE
_SC_ROWS = 8704
_TC_ROWS = _ROWS - _SC_ROWS

_K = 4
_NW = 32
_ROWS_PER_W = _SC_ROWS // _NW
_OUTER = _ROWS_PER_W // _K


def _sc_body(x_hbm, table_hbm, out_hbm, idx_v, rows_v, table_s,
             sem_idx, sem_g, sem_out):
    wid = lax.axis_index("s") * 2 + lax.axis_index("c")
    w_base = wid * _ROWS_PER_W

    @pl.when(lax.axis_index("s") == 0)
    def _():
        pltpu.sync_copy(table_hbm, table_s)
    plsc.subcore_barrier()

    def idx_copy(slot, base):
        return pltpu.make_async_copy(
            x_hbm.at[pl.ds(base, _K)], idx_v.at[slot], sem_idx)

    def out_copy(slot, base):
        return pltpu.make_async_copy(
            rows_v.at[slot], out_hbm.at[pl.ds(base, _K)], sem_out)

    idx_copy(0, w_base).start()
    idx_copy(1, w_base + _K).start()

    def body(i, carry):
        for u in range(4):
            it = i * 4 + u
            rb = u % 2
            sb = u
            base = w_base + it * _K
            idx_copy(sb, base).wait()

            @pl.when(it >= 2)
            def _():
                out_copy(rb, base - 2 * _K).wait()

            gathers = [
                pltpu.async_copy(
                    table_s.at[idx_v.at[sb, j]], rows_v.at[rb, j], sem_g)
                for j in range(_K)
            ]

            @pl.when(it + 2 < _OUTER)
            def _():
                idx_copy((u + 2) % 4, base + 2 * _K).start()

            for g in gathers:
                g.wait()
            out_copy(rb, base).start()
        return carry

    lax.fori_loop(0, _OUTER // 4, body, 0)
    out_copy(0, w_base + (_OUTER - 2) * _K).wait()
    out_copy(1, w_base + (_OUTER - 1) * _K).wait()


def _run_sc(x2, table):
    mesh = plsc.VectorSubcoreMesh(core_axis_name="c", subcore_axis_name="s")
    return functools.partial(
        pl.kernel,
        mesh=mesh,
        out_type=jax.ShapeDtypeStruct((_ROWS, _LANE, HIDDEN_DIM),
                                      jnp.float32),
        scratch_types=[
            pltpu.VMEM((4, _K, _LANE), jnp.int32),
            pltpu.VMEM((2, _K, _LANE, HIDDEN_DIM), jnp.float32),
            pltpu.VMEM_SHARED((VOCAB_SIZE, HIDDEN_DIM), jnp.float32),
            pltpu.SemaphoreType.DMA,
            pltpu.SemaphoreType.DMA,
            pltpu.SemaphoreType.DMA,
        ],
        compiler_params=pltpu.CompilerParams(use_tc_tiling_on_sc=False),
    )(_sc_body)(x2, table)


_TL = 1024
_SR = 32
_TC_N = _TC_ROWS * _LANE
_TC_XROWS = _TC_N // _TL


_TC_OFF = _SC_ROWS * _LANE // (_SR * _TL)


def _tc_body(x_ref, hi_ref, buf_ref, o_ref):
    dn = (((0,), (0,)), ((), ()))
    iota = jax.lax.broadcasted_iota(jnp.int32, (VOCAB_SIZE, _TL), 0)
    for j in range(_SR):
        oh = (x_ref[j, :][None, :] == iota).astype(jnp.bfloat16)
        acc = jax.lax.dot_general(oh, hi_ref[...], dn,
                                  preferred_element_type=jnp.float32)
        o_ref[pl.ds(j * _TL, _TL), :] = acc


def _run_tc(x2, table, buf):
    xf = x2.reshape(_TC_XROWS, _TL)
    hi = table.astype(jnp.bfloat16)
    return pl.pallas_call(
        _tc_body,
        grid=(_TC_XROWS // _SR,),
        in_specs=[
            pl.BlockSpec((_SR, _TL), lambda i: (i, 0)),
            pl.BlockSpec((VOCAB_SIZE, HIDDEN_DIM), lambda i: (0, 0)),
            pl.BlockSpec(memory_space=pltpu.MemorySpace.HBM),
        ],
        out_specs=pl.BlockSpec((_SR * _TL, HIDDEN_DIM),
                               lambda i: (i + _TC_OFF, 0)),
        out_shape=jax.ShapeDtypeStruct((_ROWS * _LANE, HIDDEN_DIM),
                                       jnp.float32),
        input_output_aliases={2: 0},
    )(xf, hi, buf)


def kernel(x, table):
    x2 = x.reshape(_ROWS, _LANE).astype(jnp.int32)
    buf = _run_sc(x2[:_SC_ROWS], table)
    out = _run_tc(x2[_SC_ROWS:], table,
                  buf.reshape(_ROWS * _LANE, HIDDEN_DIM))
    return out.reshape(16384, 200, HIDDEN_DIM)
